# Initial kernel scaffold; baseline (speedup 1.0000x reference)
import jax
import jax.numpy as jnp
from jax.experimental import pallas as pl


def kernel(input_sequence, t_representation, tables):
    raise NotImplementedError("write your pallas kernel here")


# speedup vs baseline: 1.8845x; 1.8845x over previous
import functools

import jax
import jax.numpy as jnp
from jax import lax
from jax.experimental import pallas as pl
from jax.experimental.pallas import tpu as pltpu
from jax.experimental.pallas import tpu_sc as plsc

N_DIGITS = 8
N_ARY_IN = 256
N_ARY_OUT = 64
N_DIM_EMB = 64
BATCH = 1024
ROW = N_ARY_OUT * N_DIM_EMB

NC = 2
NS = 16
NW = NC * NS
BPW = BATCH // NW


def _make_sc_call():
    mesh = plsc.VectorSubcoreMesh(core_axis_name="c", subcore_axis_name="s")

    @functools.partial(
        pl.kernel,
        mesh=mesh,
        out_type=jax.ShapeDtypeStruct((BATCH * N_DIGITS * N_ARY_OUT,), jnp.float32),
        scratch_types=[
            pltpu.VMEM((BPW, N_DIGITS), jnp.int32),
            pltpu.VMEM((BPW * N_DIGITS * N_DIM_EMB,), jnp.float32),
            pltpu.VMEM((N_DIGITS, ROW), jnp.float32),
            pltpu.VMEM((ROW,), jnp.float32),
            pltpu.VMEM((BPW * N_DIGITS * N_ARY_OUT,), jnp.float32),
            pltpu.SemaphoreType.DMA,
        ],
    )
    def sc_kernel(idx_hbm, rep_hbm, tab_hbm, out_hbm,
                  idx_v, r_v, rows_v, acc_v, out_v, sem):
        wid = lax.axis_index("s") * NC + lax.axis_index("c")
        b0 = wid * BPW
        pltpu.sync_copy(idx_hbm.at[pl.ds(b0, BPW)], idx_v)
        pltpu.sync_copy(rep_hbm.at[pl.ds(b0 * N_DIGITS * N_DIM_EMB,
                                         BPW * N_DIGITS * N_DIM_EMB)], r_v)
        lane = lax.iota(jnp.int32, 16)
        masks = [lane == j for j in range(16)]
        perms = [lane ^ k for k in (8, 4, 2, 1)]

        def lane_sum(v):
            for p in perms:
                v = v + v.at[p].get(mode="promise_in_bounds")
            return v

        def b_body(b, _):
            pltpu.async_copy(tab_hbm.at[idx_v.at[b]], rows_v, sem).wait()
            for d in range(N_DIGITS):
                base = (b * N_DIGITS + d) * N_DIM_EMB
                inv = jnp.float32(1.0 / (d + 1))
                rch = [r_v[pl.ds(base + m * 16, 16)] * inv for m in range(4)]

                def ob_body(ob, _, d=d, rch=rch, base=base):
                    ov = jnp.zeros((16,), jnp.float32)
                    for j in range(16):
                        off = (ob * 16 + j) * N_DIM_EMB
                        f = None
                        for m in range(4):
                            chunk = rows_v[d, pl.ds(off + m * 16, 16)]
                            if d > 0:
                                chunk = acc_v[pl.ds(off + m * 16, 16)] + chunk
                            acc_v[pl.ds(off + m * 16, 16)] = chunk
                            p = chunk * rch[m]
                            f = p if f is None else f + p
                        ov = jnp.where(masks[j], lane_sum(f), ov)
                    out_v[pl.ds(base + ob * 16, 16)] = ov
                    return _

                lax.fori_loop(0, 4, ob_body, None)
            return _

        lax.fori_loop(0, BPW, b_body, None)
        pltpu.sync_copy(out_v, out_hbm.at[pl.ds(b0 * N_DIGITS * N_ARY_OUT,
                                                BPW * N_DIGITS * N_ARY_OUT)])

    return sc_kernel


_SC_CALL = _make_sc_call()


def kernel(input_sequence, t_representation, tables):
    B, D = input_sequence.shape
    flat_idx = input_sequence + (jnp.arange(D, dtype=jnp.int32) * N_ARY_IN)[None, :]
    tab2 = tables.reshape(D * N_ARY_IN, ROW)
    rep_flat = t_representation.reshape(-1)
    out_flat = _SC_CALL(flat_idx, rep_flat, tab2)
    return out_flat.reshape(B, D, N_ARY_OUT)
